# baseline (speedup 1.0000x reference)
import functools
import math

import jax
import jax.numpy as jnp
from jax import lax
from jax.experimental import pallas as pl
from jax.experimental.pallas import tpu as pltpu
from jax.experimental.pallas import tpu_sc as plsc

N_NODES = 10000
N_EDGES = 320000
D_IN = 128
D_HID = 64

NC = 2
NS = 16
LANES = 16
NW = NC * NS

CHUNK = 125
CH_PER_TILE = 80
EPT = CH_PER_TILE * CHUNK
RPT = N_NODES // NS

_MESH = plsc.VectorSubcoreMesh(
    core_axis_name="c", subcore_axis_name="s", num_cores=NC, num_subcores=NS
)
_CP = pltpu.CompilerParams(use_tc_tiling_on_sc=False, needs_layout_passes=False)


@functools.partial(
    pl.kernel,
    out_type=jax.ShapeDtypeStruct((NC, N_NODES, 8), jnp.float32),
    mesh=_MESH,
    compiler_params=_CP,
    scratch_types=[
        pltpu.VMEM((CH_PER_TILE, CHUNK), jnp.int32),
        pltpu.VMEM((CHUNK, 8), jnp.float32),
        pltpu.VMEM_SHARED((N_NODES, 8), jnp.float32),
    ],
)
def _deg_count(dstr_hbm, zeros8_hbm, ones_hbm, out_hbm, dst_v, ones_v, deg_sh):
    c = lax.axis_index("c")
    s = lax.axis_index("s")
    wid = c * NS + s
    pltpu.sync_copy(zeros8_hbm.at[pl.ds(s * RPT, RPT)], deg_sh.at[pl.ds(s * RPT, RPT)])
    pltpu.sync_copy(dstr_hbm.at[pl.ds(wid * CH_PER_TILE, CH_PER_TILE)], dst_v)
    pltpu.sync_copy(ones_hbm, ones_v)
    plsc.subcore_barrier()

    def body(j, carry):
        pltpu.sync_copy(ones_v, deg_sh.at[dst_v.at[j]], add=True)
        return carry

    lax.fori_loop(0, CH_PER_TILE, body, 0)
    plsc.subcore_barrier()
    pltpu.sync_copy(deg_sh.at[pl.ds(s * RPT, RPT)], out_hbm.at[c].at[pl.ds(s * RPT, RPT)])


_GRP = 4
_NGRP = CH_PER_TILE // _GRP
_RUN = 25


@functools.partial(
    pl.kernel,
    out_type=jax.ShapeDtypeStruct((NC, N_NODES, D_HID), jnp.float32),
    mesh=_MESH,
    compiler_params=_CP,
    scratch_types=[
        pltpu.VMEM((CH_PER_TILE, CHUNK), jnp.int32),
        pltpu.VMEM((CH_PER_TILE, CHUNK), jnp.int32),
        pltpu.VMEM((2 * _GRP, CHUNK, D_HID), jnp.bfloat16),
        pltpu.VMEM((_GRP, CHUNK, D_HID), jnp.float32),
        pltpu.VMEM_SHARED((N_NODES, D_HID), jnp.float32),
        pltpu.SemaphoreType.DMA,
        pltpu.SemaphoreType.DMA,
    ],
)
def _conv_scatter(hs_hbm, srcr_hbm, dstr_hbm, zeros64_hbm, out_hbm,
                  src_v, dst_v, gbuf, sbuf, u_sh, sem, sem_s):
    c = lax.axis_index("c")
    s = lax.axis_index("s")
    wid = c * NS + s
    row0 = wid * CH_PER_TILE
    pltpu.sync_copy(srcr_hbm.at[pl.ds(row0, CH_PER_TILE)], src_v)
    pltpu.sync_copy(dstr_hbm.at[pl.ds(row0, CH_PER_TILE)], dst_v)
    pltpu.sync_copy(zeros64_hbm.at[pl.ds(s * RPT, RPT)], u_sh.at[pl.ds(s * RPT, RPT)])
    plsc.subcore_barrier()

    def fire(t):
        base = (t % 2) * _GRP
        for b in range(_GRP):
            pltpu.async_copy(hs_hbm.at[src_v.at[t * _GRP + b]], gbuf.at[base + b], sem)

    def drain_scatters(t):
        for b in range(_GRP):
            pltpu.make_async_copy(
                sbuf.at[b], u_sh.at[dst_v.at[t * _GRP + b]], sem_s
            ).wait()

    fire(0)

    def group(t, carry):
        @pl.when(t >= 1)
        def _():
            drain_scatters(t - 1)

        @pl.when(t + 1 < _NGRP)
        def _():
            fire(t + 1)

        base = (t % 2) * _GRP
        for b in range(_GRP):
            pltpu.make_async_copy(
                hs_hbm.at[src_v.at[t * _GRP + b]], gbuf.at[base + b], sem
            ).wait()

            for r in range(CHUNK):
                for q in range(D_HID // 32):
                    v = gbuf[base + b, r, pl.ds(q * 32, 32)]
                    e0, e1 = plsc.unpack(v, format=plsc.PackFormat.INTERLEAVED)
                    sbuf[b, r, pl.ds(q * 32, LANES)] = e0
                    sbuf[b, r, pl.ds(q * 32 + LANES, LANES)] = e1
            pltpu.async_copy(sbuf.at[b], u_sh.at[dst_v.at[t * _GRP + b]],
                             sem_s, add=True)
        return carry

    lax.fori_loop(0, _NGRP, group, 0)
    drain_scatters(_NGRP - 1)
    plsc.subcore_barrier()
    pltpu.sync_copy(u_sh.at[pl.ds(s * RPT, RPT)], out_hbm.at[c].at[pl.ds(s * RPT, RPT)])


_DGRP = 2
_DNGRP = CH_PER_TILE // _DGRP
_NE16 = (CHUNK + LANES - 1) // LANES
_APAD = 16


@functools.partial(
    pl.kernel,
    out_type=jax.ShapeDtypeStruct((N_EDGES,), jnp.float32),
    mesh=_MESH,
    compiler_params=_CP,
    scratch_types=[
        pltpu.VMEM((CH_PER_TILE, CHUNK), jnp.int32),
        pltpu.VMEM((CH_PER_TILE, CHUNK), jnp.int32),
        pltpu.VMEM((2 * _DGRP, CHUNK, D_HID), jnp.bfloat16),
        pltpu.VMEM((2 * _DGRP, CHUNK, D_HID), jnp.bfloat16),
        pltpu.VMEM((LANES, 17), jnp.float32),
        pltpu.VMEM((EPT + _APAD,), jnp.float32),
        pltpu.VMEM_SHARED((N_NODES, D_HID), jnp.bfloat16),
        pltpu.SemaphoreType.DMA,
        pltpu.SemaphoreType.DMA,
    ],
)
def _decode(z_hbm, srcr_hbm, dstr_hbm, out_hbm,
            src_v, dst_v, buf_a, buf_b, stage, acc_v, z_sh, sem_a, sem_b):
    c = lax.axis_index("c")
    s = lax.axis_index("s")
    wid = c * NS + s
    row0 = wid * CH_PER_TILE
    pltpu.sync_copy(srcr_hbm.at[pl.ds(row0, CH_PER_TILE)], src_v)
    pltpu.sync_copy(dstr_hbm.at[pl.ds(row0, CH_PER_TILE)], dst_v)
    pltpu.sync_copy(z_hbm.at[pl.ds(s * RPT, RPT)], z_sh.at[pl.ds(s * RPT, RPT)])
    plsc.subcore_barrier()
    lane = lax.iota(jnp.int32, LANES)

    def fire(t):
        base = (t % 2) * _DGRP
        for b in range(_DGRP):
            j = t * _DGRP + b
            pltpu.async_copy(z_sh.at[src_v.at[j]], buf_a.at[base + b], sem_a)
            pltpu.async_copy(z_sh.at[dst_v.at[j]], buf_b.at[base + b], sem_b)

    fire(0)

    def group(t, carry):
        @pl.when(t + 1 < _DNGRP)
        def _():
            fire(t + 1)

        base = (t % 2) * _DGRP
        for b in range(_DGRP):
            j = t * _DGRP + b
            pltpu.make_async_copy(z_sh.at[src_v.at[j]], buf_a.at[base + b], sem_a).wait()
            pltpu.make_async_copy(z_sh.at[dst_v.at[j]], buf_b.at[base + b], sem_b).wait()

            def grp(g, carry2, _b=base + b, _j=j):
                base_e = g * LANES
                for e in range(LANES):
                    row = base_e + e
                    sv = None
                    for q in range(D_HID // 32):
                        av = buf_a[_b, row, pl.ds(q * 32, 32)]
                        bv = buf_b[_b, row, pl.ds(q * 32, 32)]
                        a0, a1 = plsc.unpack(av, format=plsc.PackFormat.INTERLEAVED)
                        b0, b1 = plsc.unpack(bv, format=plsc.PackFormat.INTERLEAVED)
                        p = a0 * b0 + a1 * b1
                        sv = p if sv is None else sv + p
                    stage[e, pl.ds(0, LANES)] = sv
                out16 = jnp.zeros((LANES,), jnp.float32)
                for l in range(LANES):
                    cols = jnp.full((LANES,), l, jnp.int32)
                    out16 = out16 + plsc.load_gather(stage, [lane, cols])
                acc_v[pl.ds(_j * CHUNK + base_e, LANES)] = out16
                return carry2

            for g in range(_NE16):
                grp(g, 0)
        return carry

    lax.fori_loop(0, _DNGRP, group, 0)
    pltpu.sync_copy(acc_v.at[pl.ds(0, EPT)], out_hbm.at[pl.ds(wid * EPT, EPT)])


_R = 1000


def _tc_enc0(x, degp, w0):
    def body(x_ref, deg_ref, w_ref, h0_ref, hs0_ref):
        deg = (deg_ref[0] + deg_ref[1])[:, :1] + 1.0
        dinv = lax.rsqrt(deg)
        h0 = jnp.dot(x_ref[...], w_ref[...], preferred_element_type=jnp.float32)
        h0_ref[...] = h0
        hs0_ref[...] = h0 * dinv

    return pl.pallas_call(
        body,
        grid=(N_NODES // _R,),
        in_specs=[
            pl.BlockSpec((_R, D_IN), lambda i: (i, 0)),
            pl.BlockSpec((NC, _R, 8), lambda i: (0, i, 0)),
            pl.BlockSpec((D_IN, D_HID), lambda i: (0, 0)),
        ],
        out_specs=[
            pl.BlockSpec((_R, D_HID), lambda i: (i, 0)),
            pl.BlockSpec((_R, D_HID), lambda i: (i, 0)),
        ],
        out_shape=[
            jax.ShapeDtypeStruct((N_NODES, D_HID), jnp.float32),
            jax.ShapeDtypeStruct((N_NODES, D_HID), jnp.float32),
        ],
    )(x, degp, w0)


def _tc_mid(u0p, h0, degp, w1, b0, gamma, beta):
    gs = 1.0 / math.sqrt(1.0 + 1e-5)

    def body(u_ref, h0_ref, deg_ref, w_ref, b0_ref, g_ref, be_ref, h1_ref, hs1_ref):
        deg = (deg_ref[0] + deg_ref[1])[:, :1] + 1.0
        dinv = lax.rsqrt(deg)
        u = u_ref[0] + u_ref[1]
        out0 = dinv * u + (dinv * dinv) * h0_ref[...] + b0_ref[...]
        hbn = jnp.maximum(out0 * (g_ref[...] * gs) + be_ref[...], 0.0)
        h1 = jnp.dot(hbn, w_ref[...], preferred_element_type=jnp.float32)
        h1_ref[...] = h1
        hs1_ref[...] = h1 * dinv

    return pl.pallas_call(
        body,
        grid=(N_NODES // _R,),
        in_specs=[
            pl.BlockSpec((NC, _R, D_HID), lambda i: (0, i, 0)),
            pl.BlockSpec((_R, D_HID), lambda i: (i, 0)),
            pl.BlockSpec((NC, _R, 8), lambda i: (0, i, 0)),
            pl.BlockSpec((D_HID, D_HID), lambda i: (0, 0)),
            pl.BlockSpec((1, D_HID), lambda i: (0, 0)),
            pl.BlockSpec((1, D_HID), lambda i: (0, 0)),
            pl.BlockSpec((1, D_HID), lambda i: (0, 0)),
        ],
        out_specs=[
            pl.BlockSpec((_R, D_HID), lambda i: (i, 0)),
            pl.BlockSpec((_R, D_HID), lambda i: (i, 0)),
        ],
        out_shape=[
            jax.ShapeDtypeStruct((N_NODES, D_HID), jnp.float32),
            jax.ShapeDtypeStruct((N_NODES, D_HID), jnp.float32),
        ],
    )(u0p, h0, degp, w1, b0, gamma, beta)


def _tc_fin(u1p, h1, degp, b1):
    def body(u_ref, h1_ref, deg_ref, b1_ref, z_ref):
        deg = (deg_ref[0] + deg_ref[1])[:, :1] + 1.0
        dinv = lax.rsqrt(deg)
        u = u_ref[0] + u_ref[1]
        z = dinv * u + (dinv * dinv) * h1_ref[...] + b1_ref[...]
        z_ref[...] = z.astype(jnp.bfloat16)

    return pl.pallas_call(
        body,
        grid=(N_NODES // _R,),
        in_specs=[
            pl.BlockSpec((NC, _R, D_HID), lambda i: (0, i, 0)),
            pl.BlockSpec((_R, D_HID), lambda i: (i, 0)),
            pl.BlockSpec((NC, _R, 8), lambda i: (0, i, 0)),
            pl.BlockSpec((1, D_HID), lambda i: (0, 0)),
        ],
        out_specs=pl.BlockSpec((_R, D_HID), lambda i: (i, 0)),
        out_shape=jax.ShapeDtypeStruct((N_NODES, D_HID), jnp.bfloat16),
    )(u1p, h1, degp, b1)


def _swz(a):
    ab = a.astype(jnp.bfloat16).reshape(-1, 2, 2, LANES)
    return jnp.transpose(ab, (0, 1, 3, 2)).reshape(-1, D_HID)


@jax.jit
def kernel(x, edge_index, W0, b0, W1, b1, gamma, beta):
    srcr = edge_index[0].reshape(NW * CH_PER_TILE, CHUNK)
    dstr = edge_index[1].reshape(NW * CH_PER_TILE, CHUNK)
    zeros8 = jnp.zeros((N_NODES, 8), jnp.float32)
    ones_rows = jnp.ones((CHUNK, 8), jnp.float32)
    zeros64 = jnp.zeros((N_NODES, D_HID), jnp.float32)

    degp = _deg_count(dstr, zeros8, ones_rows)
    h0, hs0 = _tc_enc0(x, degp, W0)
    u0p = _conv_scatter(_swz(hs0), srcr, dstr, zeros64)
    h1, hs1 = _tc_mid(u0p, h0, degp, W1,
                      b0.reshape(1, -1), gamma.reshape(1, -1), beta.reshape(1, -1))
    u1p = _conv_scatter(_swz(hs1), srcr, dstr, zeros64)
    z = _tc_fin(u1p, h1, degp, b1.reshape(1, -1))
    return _decode(_swz(z), srcr, dstr)


# speedup vs baseline: 1.2871x; 1.2871x over previous
import functools
import math

import jax
import jax.numpy as jnp
from jax import lax
from jax.experimental import pallas as pl
from jax.experimental.pallas import tpu as pltpu
from jax.experimental.pallas import tpu_sc as plsc

N_NODES = 10000
N_EDGES = 320000
D_IN = 128
D_HID = 64

NC = 2
NS = 16
LANES = 16
NW = NC * NS

CHUNK = 125
CH_PER_TILE = 80
EPT = CH_PER_TILE * CHUNK
RPT = N_NODES // NS

_MESH = plsc.VectorSubcoreMesh(
    core_axis_name="c", subcore_axis_name="s", num_cores=NC, num_subcores=NS
)
_CP = pltpu.CompilerParams(use_tc_tiling_on_sc=False, needs_layout_passes=False)


@functools.partial(
    pl.kernel,
    out_type=jax.ShapeDtypeStruct((NC, N_NODES, 8), jnp.float32),
    mesh=_MESH,
    compiler_params=_CP,
    scratch_types=[
        pltpu.VMEM((CH_PER_TILE, CHUNK), jnp.int32),
        pltpu.VMEM((CHUNK, 8), jnp.float32),
        pltpu.VMEM_SHARED((N_NODES, 8), jnp.float32),
    ],
)
def _deg_count(dstr_hbm, zeros8_hbm, ones_hbm, out_hbm, dst_v, ones_v, deg_sh):
    c = lax.axis_index("c")
    s = lax.axis_index("s")
    wid = c * NS + s
    pltpu.sync_copy(zeros8_hbm.at[pl.ds(s * RPT, RPT)], deg_sh.at[pl.ds(s * RPT, RPT)])
    pltpu.sync_copy(dstr_hbm.at[pl.ds(wid * CH_PER_TILE, CH_PER_TILE)], dst_v)
    pltpu.sync_copy(ones_hbm, ones_v)
    plsc.subcore_barrier()

    def body(j, carry):
        pltpu.sync_copy(ones_v, deg_sh.at[dst_v.at[j]], add=True)
        return carry

    lax.fori_loop(0, CH_PER_TILE, body, 0)
    plsc.subcore_barrier()
    pltpu.sync_copy(deg_sh.at[pl.ds(s * RPT, RPT)], out_hbm.at[c].at[pl.ds(s * RPT, RPT)])


_GRP = 4
_NGRP = CH_PER_TILE // _GRP
_RUN = 25


@functools.partial(
    pl.kernel,
    out_type=jax.ShapeDtypeStruct((NC, N_NODES, D_HID), jnp.float32),
    mesh=_MESH,
    compiler_params=_CP,
    scratch_types=[
        pltpu.VMEM((CH_PER_TILE, CHUNK), jnp.int32),
        pltpu.VMEM((CH_PER_TILE, CHUNK), jnp.int32),
        pltpu.VMEM((2 * _GRP, CHUNK, D_HID), jnp.bfloat16),
        pltpu.VMEM((_GRP, CHUNK, D_HID), jnp.float32),
        pltpu.VMEM_SHARED((N_NODES, D_HID), jnp.float32),
        pltpu.SemaphoreType.DMA,
        pltpu.SemaphoreType.DMA,
    ],
)
def _conv_scatter(hs_hbm, srcr_hbm, dstr_hbm, zeros64_hbm, out_hbm,
                  src_v, dst_v, gbuf, sbuf, u_sh, sem, sem_s):
    c = lax.axis_index("c")
    s = lax.axis_index("s")
    wid = c * NS + s
    row0 = wid * CH_PER_TILE
    pltpu.sync_copy(srcr_hbm.at[pl.ds(row0, CH_PER_TILE)], src_v)
    pltpu.sync_copy(dstr_hbm.at[pl.ds(row0, CH_PER_TILE)], dst_v)
    pltpu.sync_copy(zeros64_hbm.at[pl.ds(s * RPT, RPT)], u_sh.at[pl.ds(s * RPT, RPT)])
    plsc.subcore_barrier()

    def fire(t):
        base = (t % 2) * _GRP
        for b in range(_GRP):
            pltpu.async_copy(hs_hbm.at[src_v.at[t * _GRP + b]], gbuf.at[base + b], sem)

    def drain_scatters(t):
        for b in range(_GRP):
            pltpu.make_async_copy(
                sbuf.at[b], u_sh.at[dst_v.at[t * _GRP + b]], sem_s
            ).wait()

    fire(0)

    def group(t, carry):
        @pl.when(t >= 1)
        def _():
            drain_scatters(t - 1)

        @pl.when(t + 1 < _NGRP)
        def _():
            fire(t + 1)

        base = (t % 2) * _GRP
        for b in range(_GRP):
            pltpu.make_async_copy(
                hs_hbm.at[src_v.at[t * _GRP + b]], gbuf.at[base + b], sem
            ).wait()

            def conv_rows(i, carry2, _g=base + b, _s=b):
                r0 = i * _RUN
                for dr in range(_RUN):
                    r = r0 + dr
                    for q in range(D_HID // 32):
                        v = gbuf[_g, r, pl.ds(q * 32, 32)]
                        e0, e1 = plsc.unpack(v, format=plsc.PackFormat.INTERLEAVED)
                        sbuf[_s, r, pl.ds(q * 32, LANES)] = e0
                        sbuf[_s, r, pl.ds(q * 32 + LANES, LANES)] = e1
                return carry2

            lax.fori_loop(0, CHUNK // _RUN, conv_rows, 0)
            pltpu.async_copy(sbuf.at[b], u_sh.at[dst_v.at[t * _GRP + b]],
                             sem_s, add=True)
        return carry

    lax.fori_loop(0, _NGRP, group, 0)
    drain_scatters(_NGRP - 1)
    plsc.subcore_barrier()
    pltpu.sync_copy(u_sh.at[pl.ds(s * RPT, RPT)], out_hbm.at[c].at[pl.ds(s * RPT, RPT)])


_DGRP = 4
_DNGRP = CH_PER_TILE // _DGRP
_NE16 = (CHUNK + LANES - 1) // LANES
_APAD = 16


@functools.partial(
    pl.kernel,
    out_type=jax.ShapeDtypeStruct((N_EDGES,), jnp.float32),
    mesh=_MESH,
    compiler_params=_CP,
    scratch_types=[
        pltpu.VMEM((CH_PER_TILE, CHUNK), jnp.int32),
        pltpu.VMEM((CH_PER_TILE, CHUNK), jnp.int32),
        pltpu.VMEM((2 * _DGRP, CHUNK, D_HID), jnp.bfloat16),
        pltpu.VMEM((2 * _DGRP, CHUNK, D_HID), jnp.bfloat16),
        pltpu.VMEM((LANES, 17), jnp.float32),
        pltpu.VMEM((EPT + _APAD,), jnp.float32),
        pltpu.VMEM_SHARED((N_NODES, D_HID), jnp.bfloat16),
        pltpu.SemaphoreType.DMA,
        pltpu.SemaphoreType.DMA,
    ],
)
def _decode(z_hbm, srcr_hbm, dstr_hbm, out_hbm,
            src_v, dst_v, buf_a, buf_b, stage, acc_v, z_sh, sem_a, sem_b):
    c = lax.axis_index("c")
    s = lax.axis_index("s")
    wid = c * NS + s
    row0 = wid * CH_PER_TILE
    pltpu.sync_copy(srcr_hbm.at[pl.ds(row0, CH_PER_TILE)], src_v)
    pltpu.sync_copy(dstr_hbm.at[pl.ds(row0, CH_PER_TILE)], dst_v)
    pltpu.sync_copy(z_hbm.at[pl.ds(s * RPT, RPT)], z_sh.at[pl.ds(s * RPT, RPT)])
    plsc.subcore_barrier()
    lane = lax.iota(jnp.int32, LANES)

    def fire(t):
        base = (t % 2) * _DGRP
        for b in range(_DGRP):
            j = t * _DGRP + b
            pltpu.async_copy(z_sh.at[src_v.at[j]], buf_a.at[base + b], sem_a)
            pltpu.async_copy(z_sh.at[dst_v.at[j]], buf_b.at[base + b], sem_b)

    fire(0)

    def group(t, carry):
        @pl.when(t + 1 < _DNGRP)
        def _():
            fire(t + 1)

        base = (t % 2) * _DGRP
        for b in range(_DGRP):
            j = t * _DGRP + b
            pltpu.make_async_copy(z_sh.at[src_v.at[j]], buf_a.at[base + b], sem_a).wait()
            pltpu.make_async_copy(z_sh.at[dst_v.at[j]], buf_b.at[base + b], sem_b).wait()

            def grp(g, carry2, _b=base + b, _j=j):
                base_e = g * LANES
                for e in range(LANES):
                    row = base_e + e
                    sv = None
                    for q in range(D_HID // 32):
                        av = buf_a[_b, row, pl.ds(q * 32, 32)]
                        bv = buf_b[_b, row, pl.ds(q * 32, 32)]
                        a0, a1 = plsc.unpack(av, format=plsc.PackFormat.INTERLEAVED)
                        b0, b1 = plsc.unpack(bv, format=plsc.PackFormat.INTERLEAVED)
                        p = a0 * b0 + a1 * b1
                        sv = p if sv is None else sv + p
                    stage[e, pl.ds(0, LANES)] = sv
                out16 = jnp.zeros((LANES,), jnp.float32)
                for l in range(LANES):
                    cols = jnp.full((LANES,), l, jnp.int32)
                    out16 = out16 + plsc.load_gather(stage, [lane, cols])
                acc_v[pl.ds(_j * CHUNK + base_e, LANES)] = out16
                return carry2

            lax.fori_loop(0, _NE16, grp, 0)
        return carry

    lax.fori_loop(0, _DNGRP, group, 0)
    pltpu.sync_copy(acc_v.at[pl.ds(0, EPT)], out_hbm.at[pl.ds(wid * EPT, EPT)])


_R = 1000


def _tc_enc0(x, degp, w0):
    def body(x_ref, deg_ref, w_ref, h0_ref, hs0_ref):
        deg = (deg_ref[0] + deg_ref[1])[:, :1] + 1.0
        dinv = lax.rsqrt(deg)
        h0 = jnp.dot(x_ref[...], w_ref[...], preferred_element_type=jnp.float32)
        h0_ref[...] = h0
        hs0_ref[...] = h0 * dinv

    return pl.pallas_call(
        body,
        grid=(N_NODES // _R,),
        in_specs=[
            pl.BlockSpec((_R, D_IN), lambda i: (i, 0)),
            pl.BlockSpec((NC, _R, 8), lambda i: (0, i, 0)),
            pl.BlockSpec((D_IN, D_HID), lambda i: (0, 0)),
        ],
        out_specs=[
            pl.BlockSpec((_R, D_HID), lambda i: (i, 0)),
            pl.BlockSpec((_R, D_HID), lambda i: (i, 0)),
        ],
        out_shape=[
            jax.ShapeDtypeStruct((N_NODES, D_HID), jnp.float32),
            jax.ShapeDtypeStruct((N_NODES, D_HID), jnp.float32),
        ],
    )(x, degp, w0)


def _tc_mid(u0p, h0, degp, w1, b0, gamma, beta):
    gs = 1.0 / math.sqrt(1.0 + 1e-5)

    def body(u_ref, h0_ref, deg_ref, w_ref, b0_ref, g_ref, be_ref, h1_ref, hs1_ref):
        deg = (deg_ref[0] + deg_ref[1])[:, :1] + 1.0
        dinv = lax.rsqrt(deg)
        u = u_ref[0] + u_ref[1]
        out0 = dinv * u + (dinv * dinv) * h0_ref[...] + b0_ref[...]
        hbn = jnp.maximum(out0 * (g_ref[...] * gs) + be_ref[...], 0.0)
        h1 = jnp.dot(hbn, w_ref[...], preferred_element_type=jnp.float32)
        h1_ref[...] = h1
        hs1_ref[...] = h1 * dinv

    return pl.pallas_call(
        body,
        grid=(N_NODES // _R,),
        in_specs=[
            pl.BlockSpec((NC, _R, D_HID), lambda i: (0, i, 0)),
            pl.BlockSpec((_R, D_HID), lambda i: (i, 0)),
            pl.BlockSpec((NC, _R, 8), lambda i: (0, i, 0)),
            pl.BlockSpec((D_HID, D_HID), lambda i: (0, 0)),
            pl.BlockSpec((1, D_HID), lambda i: (0, 0)),
            pl.BlockSpec((1, D_HID), lambda i: (0, 0)),
            pl.BlockSpec((1, D_HID), lambda i: (0, 0)),
        ],
        out_specs=[
            pl.BlockSpec((_R, D_HID), lambda i: (i, 0)),
            pl.BlockSpec((_R, D_HID), lambda i: (i, 0)),
        ],
        out_shape=[
            jax.ShapeDtypeStruct((N_NODES, D_HID), jnp.float32),
            jax.ShapeDtypeStruct((N_NODES, D_HID), jnp.float32),
        ],
    )(u0p, h0, degp, w1, b0, gamma, beta)


def _tc_fin(u1p, h1, degp, b1):
    def body(u_ref, h1_ref, deg_ref, b1_ref, z_ref):
        deg = (deg_ref[0] + deg_ref[1])[:, :1] + 1.0
        dinv = lax.rsqrt(deg)
        u = u_ref[0] + u_ref[1]
        z = dinv * u + (dinv * dinv) * h1_ref[...] + b1_ref[...]
        z_ref[...] = z.astype(jnp.bfloat16)

    return pl.pallas_call(
        body,
        grid=(N_NODES // _R,),
        in_specs=[
            pl.BlockSpec((NC, _R, D_HID), lambda i: (0, i, 0)),
            pl.BlockSpec((_R, D_HID), lambda i: (i, 0)),
            pl.BlockSpec((NC, _R, 8), lambda i: (0, i, 0)),
            pl.BlockSpec((1, D_HID), lambda i: (0, 0)),
        ],
        out_specs=pl.BlockSpec((_R, D_HID), lambda i: (i, 0)),
        out_shape=jax.ShapeDtypeStruct((N_NODES, D_HID), jnp.bfloat16),
    )(u1p, h1, degp, b1)


def _swz(a):
    ab = a.astype(jnp.bfloat16).reshape(-1, 2, 2, LANES)
    return jnp.transpose(ab, (0, 1, 3, 2)).reshape(-1, D_HID)


@jax.jit
def kernel(x, edge_index, W0, b0, W1, b1, gamma, beta):
    srcr = edge_index[0].reshape(NW * CH_PER_TILE, CHUNK)
    dstr = edge_index[1].reshape(NW * CH_PER_TILE, CHUNK)
    zeros8 = jnp.zeros((N_NODES, 8), jnp.float32)
    ones_rows = jnp.ones((CHUNK, 8), jnp.float32)
    zeros64 = jnp.zeros((N_NODES, D_HID), jnp.float32)

    degp = _deg_count(dstr, zeros8, ones_rows)
    h0, hs0 = _tc_enc0(x, degp, W0)
    u0p = _conv_scatter(_swz(hs0), srcr, dstr, zeros64)
    h1, hs1 = _tc_mid(u0p, h0, degp, W1,
                      b0.reshape(1, -1), gamma.reshape(1, -1), beta.reshape(1, -1))
    u1p = _conv_scatter(_swz(hs1), srcr, dstr, zeros64)
    z = _tc_fin(u1p, h1, degp, b1.reshape(1, -1))
    return _decode(_swz(z), srcr, dstr)
